# baseline (speedup 1.0000x reference)
import functools

import jax
import jax.numpy as jnp
from jax.experimental import pallas as pl
from jax.experimental.pallas import tpu as pltpu

_EPS = 1e-8
_VB = 2048
_SC = 512
_SC2 = 2048


def _k1_kernel(v, nchunk0, nj1, kw_ref, wp_ref, b_ref, we_ref,
               cos_ref, s_out_ref, acc_out_ref,
               kwn_ref, acc_ref, s_ref):
    r = pl.program_id(0)
    j = pl.program_id(1)

    @pl.when(j == 0)
    def _init():
        kw = jnp.dot(kw_ref[...], wp_ref[...],
                     preferred_element_type=jnp.float32) + b_ref[...]
        nrm = jnp.sqrt(jnp.sum(kw * kw, axis=1, keepdims=True))
        kwn_ref[...] = (kw / jnp.maximum(nrm, _EPS)).astype(jnp.bfloat16)
        acc_ref[...] = jnp.zeros_like(acc_ref)
        s_ref[...] = jnp.zeros_like(s_ref)

    kwn = kwn_ref[...]
    td = we_ref.shape[1]
    base = (r * nchunk0 + j) * _VB
    lo = r * (nchunk0 * _VB)
    hi = jnp.where(r == 0, nchunk0 * _VB, v)

    ones8 = jnp.ones((8, td), jnp.float32)
    s_loc = s_ref[...]
    for c in range(_VB // _SC):
        e_c = we_ref[c * _SC:(c + 1) * _SC, :]
        rid = jax.lax.broadcasted_iota(jnp.int32, (_SC, td), 0) + (
            base + c * _SC)
        e_m = jnp.where(rid < v, e_c, 0.0)
        e_bf = e_m.astype(jnp.bfloat16)
        nsq = jax.lax.dot_general(ones8, e_m * e_m,
                                  (((1,), (1,)), ((), ())),
                                  preferred_element_type=jnp.float32)[0:1, :]
        inv_n = 1.0 / jnp.maximum(jnp.sqrt(nsq), _EPS)
        cos_c = jax.lax.dot_general(kwn, e_bf, (((1,), (1,)), ((), ())),
                                    preferred_element_type=jnp.float32)
        cos_c = cos_c * inv_n
        cos_ref[:, c * _SC:(c + 1) * _SC] = cos_c
        cid = jax.lax.broadcasted_iota(jnp.int32, (1, _SC), 1) + (
            base + c * _SC)
        p = jnp.where(cid >= lo, jnp.where(cid < hi,
                                           jnp.exp(cos_c - 1.0), 0.0), 0.0)
        s_loc = s_loc + jnp.sum(p, axis=1, keepdims=True)
        acc_ref[...] = acc_ref[...] + jnp.dot(
            p.astype(jnp.bfloat16), e_bf,
            preferred_element_type=jnp.float32)
    s_ref[...] = s_loc

    @pl.when(j == nj1 - 1)
    def _fin():
        s_out_ref[0] = s_ref[...]
        acc_out_ref[0] = acc_ref[...]


def _k2_kernel(v, cos_ref, s_ref, acc_ref, prob_ref, out_ref):
    s_tot = s_ref[0] + s_ref[1]
    inv = 1.0 / s_tot
    n_full, rem = divmod(v, _SC2)
    for c in range(n_full + (1 if rem else 0)):
        sl = slice(c * _SC2, min((c + 1) * _SC2, v))
        prob_ref[:, sl] = jnp.exp(cos_ref[:, sl] - 1.0) * inv
    out_ref[...] = (acc_ref[0] + acc_ref[1]) * inv


def kernel(keywords, W_proj, b_proj, W_emb):
    bsz, kwn, dm = keywords.shape
    v, td = W_emb.shape
    bk = bsz * kwn

    kw2 = keywords.reshape(bk, dm)
    b2 = b_proj.reshape(1, td)

    total_chunks = pl.cdiv(v, _VB)
    nchunk0 = total_chunks // 2
    nj1 = total_chunks - nchunk0

    cos, s_part, acc_part = pl.pallas_call(
        functools.partial(_k1_kernel, v, nchunk0, nj1),
        grid=(2, nj1),
        in_specs=[
            pl.BlockSpec((bk, dm), lambda r, j: (0, 0)),
            pl.BlockSpec((dm, td), lambda r, j: (0, 0)),
            pl.BlockSpec((1, td), lambda r, j: (0, 0)),
            pl.BlockSpec((_VB, td), lambda r, j: (r * nchunk0 + j, 0)),
        ],
        out_specs=[
            pl.BlockSpec((bk, _VB), lambda r, j: (0, r * nchunk0 + j)),
            pl.BlockSpec((1, bk, 1), lambda r, j: (r, 0, 0)),
            pl.BlockSpec((1, bk, td), lambda r, j: (r, 0, 0)),
        ],
        out_shape=(
            jax.ShapeDtypeStruct((bk, v), jnp.float32),
            jax.ShapeDtypeStruct((2, bk, 1), jnp.float32),
            jax.ShapeDtypeStruct((2, bk, td), jnp.float32),
        ),
        scratch_shapes=[
            pltpu.VMEM((bk, td), jnp.bfloat16),
            pltpu.VMEM((bk, td), jnp.float32),
            pltpu.VMEM((bk, 1), jnp.float32),
        ],
        compiler_params=pltpu.CompilerParams(
            dimension_semantics=("parallel", "arbitrary"),
            vmem_limit_bytes=64 * 1024 * 1024,
        ),
    )(kw2, W_proj, b2, W_emb)

    rs = min(64, bk // 2)
    n_strips = bk // rs
    prob, out = pl.pallas_call(
        functools.partial(_k2_kernel, v),
        grid=(2, n_strips // 2),
        in_specs=[
            pl.BlockSpec((rs, v),
                         lambda r, t: (r * (n_strips // 2) + t, 0)),
            pl.BlockSpec((2, rs, 1),
                         lambda r, t: (0, r * (n_strips // 2) + t, 0)),
            pl.BlockSpec((2, rs, td),
                         lambda r, t: (0, r * (n_strips // 2) + t, 0)),
        ],
        out_specs=[
            pl.BlockSpec((rs, v),
                         lambda r, t: (r * (n_strips // 2) + t, 0)),
            pl.BlockSpec((rs, td),
                         lambda r, t: (r * (n_strips // 2) + t, 0)),
        ],
        out_shape=(
            jax.ShapeDtypeStruct((bk, v), jnp.float32),
            jax.ShapeDtypeStruct((bk, td), jnp.float32),
        ),
        compiler_params=pltpu.CompilerParams(
            dimension_semantics=("parallel", "arbitrary"),
            vmem_limit_bytes=64 * 1024 * 1024,
        ),
    )(cos, s_part, acc_part)

    return (out.reshape(bsz, kwn, td),
            prob.reshape(bsz, kwn, v),
            cos.reshape(bsz, kwn, v))


# speedup vs baseline: 1.0606x; 1.0606x over previous
import functools

import jax
import jax.numpy as jnp
from jax.experimental import pallas as pl
from jax.experimental.pallas import tpu as pltpu

_EPS = 1e-8
_VB = 2048
_SC = 1024
_SC2 = 2048


def _k1_kernel(v, nchunk0, nj1, kw_ref, wp_ref, b_ref, we_ref,
               cos_ref, s_out_ref, acc_out_ref,
               kwn_ref, acc_ref, s_ref):
    r = pl.program_id(0)
    j = pl.program_id(1)

    @pl.when(j == 0)
    def _init():
        kw = jnp.dot(kw_ref[...], wp_ref[...],
                     preferred_element_type=jnp.float32) + b_ref[...]
        nrm = jnp.sqrt(jnp.sum(kw * kw, axis=1, keepdims=True))
        kwn_ref[...] = (kw / jnp.maximum(nrm, _EPS)).astype(jnp.bfloat16)
        acc_ref[...] = jnp.zeros_like(acc_ref)
        s_ref[...] = jnp.zeros_like(s_ref)

    kwn = kwn_ref[...]
    td = we_ref.shape[1]
    base = (r * nchunk0 + j) * _VB
    lo = r * (nchunk0 * _VB)
    hi = jnp.where(r == 0, nchunk0 * _VB, v)

    s_loc = s_ref[...]
    for c in range(_VB // _SC):
        e_c = we_ref[c * _SC:(c + 1) * _SC, :]
        rid = jax.lax.broadcasted_iota(jnp.int32, (_SC, td), 0) + (
            base + c * _SC)
        e_m = jnp.where(rid < v, e_c, 0.0)
        nrm = jnp.sqrt(jnp.sum(e_m * e_m, axis=1, keepdims=True))
        en = (e_m / jnp.maximum(nrm, _EPS)).astype(jnp.bfloat16)
        cos_c = jax.lax.dot_general(kwn, en, (((1,), (1,)), ((), ())),
                                    preferred_element_type=jnp.float32)
        cos_ref[:, c * _SC:(c + 1) * _SC] = cos_c
        cid = jax.lax.broadcasted_iota(jnp.int32, (1, _SC), 1) + (
            base + c * _SC)
        p = jnp.where(cid >= lo, jnp.where(cid < hi,
                                           jnp.exp(cos_c - 1.0), 0.0), 0.0)
        s_loc = s_loc + jnp.sum(p, axis=1, keepdims=True)
        acc_ref[...] = acc_ref[...] + jnp.dot(
            p.astype(jnp.bfloat16), e_m.astype(jnp.bfloat16),
            preferred_element_type=jnp.float32)
    s_ref[...] = s_loc

    @pl.when(j == nj1 - 1)
    def _fin():
        s_out_ref[0] = s_ref[...]
        acc_out_ref[0] = acc_ref[...]


def _k2_kernel(v, cos_ref, s_ref, acc_ref, prob_ref, out_ref):
    s_tot = s_ref[0] + s_ref[1]
    inv = 1.0 / s_tot
    n_full, rem = divmod(v, _SC2)
    for c in range(n_full + (1 if rem else 0)):
        sl = slice(c * _SC2, min((c + 1) * _SC2, v))
        prob_ref[:, sl] = jnp.exp(cos_ref[:, sl] - 1.0) * inv
    out_ref[...] = (acc_ref[0] + acc_ref[1]) * inv


def kernel(keywords, W_proj, b_proj, W_emb):
    bsz, kwn, dm = keywords.shape
    v, td = W_emb.shape
    bk = bsz * kwn

    kw2 = keywords.reshape(bk, dm)
    b2 = b_proj.reshape(1, td)

    total_chunks = pl.cdiv(v, _VB)
    nchunk0 = total_chunks // 2
    nj1 = total_chunks - nchunk0

    cos, s_part, acc_part = pl.pallas_call(
        functools.partial(_k1_kernel, v, nchunk0, nj1),
        grid=(2, nj1),
        in_specs=[
            pl.BlockSpec((bk, dm), lambda r, j: (0, 0)),
            pl.BlockSpec((dm, td), lambda r, j: (0, 0)),
            pl.BlockSpec((1, td), lambda r, j: (0, 0)),
            pl.BlockSpec((_VB, td), lambda r, j: (r * nchunk0 + j, 0)),
        ],
        out_specs=[
            pl.BlockSpec((bk, _VB), lambda r, j: (0, r * nchunk0 + j)),
            pl.BlockSpec((1, bk, 1), lambda r, j: (r, 0, 0)),
            pl.BlockSpec((1, bk, td), lambda r, j: (r, 0, 0)),
        ],
        out_shape=(
            jax.ShapeDtypeStruct((bk, v), jnp.float32),
            jax.ShapeDtypeStruct((2, bk, 1), jnp.float32),
            jax.ShapeDtypeStruct((2, bk, td), jnp.float32),
        ),
        scratch_shapes=[
            pltpu.VMEM((bk, td), jnp.bfloat16),
            pltpu.VMEM((bk, td), jnp.float32),
            pltpu.VMEM((bk, 1), jnp.float32),
        ],
        compiler_params=pltpu.CompilerParams(
            dimension_semantics=("parallel", "arbitrary"),
            vmem_limit_bytes=64 * 1024 * 1024,
        ),
    )(kw2, W_proj, b2, W_emb)

    rs = min(64, bk // 2)
    n_strips = bk // rs
    prob, out = pl.pallas_call(
        functools.partial(_k2_kernel, v),
        grid=(2, n_strips // 2),
        in_specs=[
            pl.BlockSpec((rs, v),
                         lambda r, t: (r * (n_strips // 2) + t, 0)),
            pl.BlockSpec((2, rs, 1),
                         lambda r, t: (0, r * (n_strips // 2) + t, 0)),
            pl.BlockSpec((2, rs, td),
                         lambda r, t: (0, r * (n_strips // 2) + t, 0)),
        ],
        out_specs=[
            pl.BlockSpec((rs, v),
                         lambda r, t: (r * (n_strips // 2) + t, 0)),
            pl.BlockSpec((rs, td),
                         lambda r, t: (r * (n_strips // 2) + t, 0)),
        ],
        out_shape=(
            jax.ShapeDtypeStruct((bk, v), jnp.float32),
            jax.ShapeDtypeStruct((bk, td), jnp.float32),
        ),
        compiler_params=pltpu.CompilerParams(
            dimension_semantics=("parallel", "arbitrary"),
            vmem_limit_bytes=64 * 1024 * 1024,
        ),
    )(cos, s_part, acc_part)

    return (out.reshape(bsz, kwn, td),
            prob.reshape(bsz, kwn, v),
            cos.reshape(bsz, kwn, v))


# speedup vs baseline: 1.1629x; 1.0965x over previous
import functools

import jax
import jax.numpy as jnp
from jax.experimental import pallas as pl
from jax.experimental.pallas import tpu as pltpu

_EPS = 1e-8
_VB = 2048
_SC = 1024
_SC2 = 2048


def _k1_kernel(v, nchunk0, nj1, kw_ref, wp_ref, b_ref, we_ref,
               cos_ref, acc_out_ref,
               kwn_ref, acc_ref):
    r = pl.program_id(0)
    j = pl.program_id(1)

    @pl.when(j == 0)
    def _init():
        kw = jnp.dot(kw_ref[...], wp_ref[...],
                     preferred_element_type=jnp.float32) + b_ref[...]
        nrm = jnp.sqrt(jnp.sum(kw * kw, axis=1, keepdims=True))
        kwn_ref[...] = (kw / jnp.maximum(nrm, _EPS)).astype(jnp.bfloat16)
        acc_ref[...] = jnp.zeros_like(acc_ref)

    kwn = kwn_ref[...]
    td = we_ref.shape[1]
    base = (r * nchunk0 + j) * _VB
    lo = r * (nchunk0 * _VB)
    hi = jnp.where(r == 0, nchunk0 * _VB, v)

    for c in range(_VB // _SC):
        e_c = we_ref[c * _SC:(c + 1) * _SC, :]
        rid = jax.lax.broadcasted_iota(jnp.int32, (_SC, td), 0) + (
            base + c * _SC)
        e_m = jnp.where(rid < v, e_c, 0.0)
        nrm = jnp.sqrt(jnp.sum(e_m * e_m, axis=1, keepdims=True))
        en = (e_m / jnp.maximum(nrm, _EPS)).astype(jnp.bfloat16)
        cos_c = jax.lax.dot_general(kwn, en, (((1,), (1,)), ((), ())),
                                    preferred_element_type=jnp.float32)
        cos_ref[:, c * _SC:(c + 1) * _SC] = cos_c
        cid = jax.lax.broadcasted_iota(jnp.int32, (1, _SC), 1) + (
            base + c * _SC)
        p = jnp.where(cid >= lo, jnp.where(cid < hi,
                                           jnp.exp(cos_c - 1.0), 0.0), 0.0)
        acc_ref[...] = acc_ref[...] + jnp.dot(
            p.astype(jnp.bfloat16), e_m.astype(jnp.bfloat16),
            preferred_element_type=jnp.float32)

    @pl.when(j == nj1 - 1)
    def _fin():
        acc_out_ref[0] = acc_ref[...]


def _k2_kernel(v, cos_ref, acc_ref, prob_ref, out_ref):
    n_chunks = -(-v // _SC2)
    s_tot = jnp.zeros((prob_ref.shape[0], 1), jnp.float32)
    for c in range(n_chunks):
        sl = slice(c * _SC2, min((c + 1) * _SC2, v))
        p = jnp.exp(cos_ref[:, sl] - 1.0)
        prob_ref[:, sl] = p
        s_tot = s_tot + jnp.sum(p, axis=1, keepdims=True)
    inv = 1.0 / s_tot
    for c in range(n_chunks):
        sl = slice(c * _SC2, min((c + 1) * _SC2, v))
        prob_ref[:, sl] = prob_ref[:, sl] * inv
    out_ref[...] = (acc_ref[0] + acc_ref[1]) * inv


def kernel(keywords, W_proj, b_proj, W_emb):
    bsz, kwn, dm = keywords.shape
    v, td = W_emb.shape
    bk = bsz * kwn

    kw2 = keywords.reshape(bk, dm)
    b2 = b_proj.reshape(1, td)

    total_chunks = pl.cdiv(v, _VB)
    nchunk0 = total_chunks // 2
    nj1 = total_chunks - nchunk0

    cos, acc_part = pl.pallas_call(
        functools.partial(_k1_kernel, v, nchunk0, nj1),
        grid=(2, nj1),
        in_specs=[
            pl.BlockSpec((bk, dm), lambda r, j: (0, 0)),
            pl.BlockSpec((dm, td), lambda r, j: (0, 0)),
            pl.BlockSpec((1, td), lambda r, j: (0, 0)),
            pl.BlockSpec((_VB, td), lambda r, j: (r * nchunk0 + j, 0)),
        ],
        out_specs=[
            pl.BlockSpec((bk, _VB), lambda r, j: (0, r * nchunk0 + j)),
            pl.BlockSpec((1, bk, td), lambda r, j: (r, 0, 0)),
        ],
        out_shape=(
            jax.ShapeDtypeStruct((bk, v), jnp.float32),
            jax.ShapeDtypeStruct((2, bk, td), jnp.float32),
        ),
        scratch_shapes=[
            pltpu.VMEM((bk, td), jnp.bfloat16),
            pltpu.VMEM((bk, td), jnp.float32),
        ],
        compiler_params=pltpu.CompilerParams(
            dimension_semantics=("parallel", "arbitrary"),
            vmem_limit_bytes=64 * 1024 * 1024,
        ),
    )(kw2, W_proj, b2, W_emb)

    rs = min(64, bk // 2)
    n_strips = bk // rs
    prob, out = pl.pallas_call(
        functools.partial(_k2_kernel, v),
        grid=(2, n_strips // 2),
        in_specs=[
            pl.BlockSpec((rs, v),
                         lambda r, t: (r * (n_strips // 2) + t, 0)),
            pl.BlockSpec((2, rs, td),
                         lambda r, t: (0, r * (n_strips // 2) + t, 0)),
        ],
        out_specs=[
            pl.BlockSpec((rs, v),
                         lambda r, t: (r * (n_strips // 2) + t, 0)),
            pl.BlockSpec((rs, td),
                         lambda r, t: (r * (n_strips // 2) + t, 0)),
        ],
        out_shape=(
            jax.ShapeDtypeStruct((bk, v), jnp.float32),
            jax.ShapeDtypeStruct((bk, td), jnp.float32),
        ),
        compiler_params=pltpu.CompilerParams(
            dimension_semantics=("parallel", "arbitrary"),
            vmem_limit_bytes=64 * 1024 * 1024,
        ),
    )(cos, acc_part)

    return (out.reshape(bsz, kwn, td),
            prob.reshape(bsz, kwn, v),
            cos.reshape(bsz, kwn, v))


# speedup vs baseline: 1.1713x; 1.0072x over previous
import functools

import jax
import jax.numpy as jnp
from jax.experimental import pallas as pl
from jax.experimental.pallas import tpu as pltpu

_EPS = 1e-8
_VB = 2048
_SC = 2048
_SC2 = 2048


def _k1_kernel(v, nchunk0, nj1, kw_ref, wp_ref, b_ref, we_ref,
               cos_ref, acc_out_ref,
               kwn_ref, acc_ref):
    r = pl.program_id(0)
    j = pl.program_id(1)

    @pl.when(j == 0)
    def _init():
        kw = jnp.dot(kw_ref[...], wp_ref[...],
                     preferred_element_type=jnp.float32) + b_ref[...]
        nrm = jnp.sqrt(jnp.sum(kw * kw, axis=1, keepdims=True))
        kwn_ref[...] = (kw / jnp.maximum(nrm, _EPS)).astype(jnp.bfloat16)
        acc_ref[...] = jnp.zeros_like(acc_ref)

    kwn = kwn_ref[...]
    td = we_ref.shape[1]
    base = (r * nchunk0 + j) * _VB
    lo = r * (nchunk0 * _VB)
    hi = jnp.where(r == 0, nchunk0 * _VB, v)

    for c in range(_VB // _SC):
        e_c = we_ref[c * _SC:(c + 1) * _SC, :]
        rid = jax.lax.broadcasted_iota(jnp.int32, (_SC, td), 0) + (
            base + c * _SC)
        e_m = jnp.where(rid < v, e_c, 0.0)
        nrm = jnp.sqrt(jnp.sum(e_m * e_m, axis=1, keepdims=True))
        en = (e_m / jnp.maximum(nrm, _EPS)).astype(jnp.bfloat16)
        cos_c = jax.lax.dot_general(kwn, en, (((1,), (1,)), ((), ())),
                                    preferred_element_type=jnp.float32)
        cos_ref[:, c * _SC:(c + 1) * _SC] = cos_c
        cid = jax.lax.broadcasted_iota(jnp.int32, (1, _SC), 1) + (
            base + c * _SC)
        p = jnp.where(cid >= lo, jnp.where(cid < hi,
                                           jnp.exp(cos_c - 1.0), 0.0), 0.0)
        acc_ref[...] = acc_ref[...] + jnp.dot(
            p.astype(jnp.bfloat16), e_m.astype(jnp.bfloat16),
            preferred_element_type=jnp.float32)

    @pl.when(j == nj1 - 1)
    def _fin():
        acc_out_ref[0] = acc_ref[...]


def _k2_kernel(v, cos_ref, acc_ref, prob_ref, out_ref):
    n_chunks = -(-v // _SC2)
    s_tot = jnp.zeros((prob_ref.shape[0], 1), jnp.float32)
    for c in range(n_chunks):
        sl = slice(c * _SC2, min((c + 1) * _SC2, v))
        p = jnp.exp(cos_ref[:, sl] - 1.0)
        prob_ref[:, sl] = p
        s_tot = s_tot + jnp.sum(p, axis=1, keepdims=True)
    inv = 1.0 / s_tot
    for c in range(n_chunks):
        sl = slice(c * _SC2, min((c + 1) * _SC2, v))
        prob_ref[:, sl] = prob_ref[:, sl] * inv
    out_ref[...] = (acc_ref[0] + acc_ref[1]) * inv


def kernel(keywords, W_proj, b_proj, W_emb):
    bsz, kwn, dm = keywords.shape
    v, td = W_emb.shape
    bk = bsz * kwn

    kw2 = keywords.reshape(bk, dm)
    b2 = b_proj.reshape(1, td)

    total_chunks = pl.cdiv(v, _VB)
    nchunk0 = total_chunks // 2
    nj1 = total_chunks - nchunk0

    cos, acc_part = pl.pallas_call(
        functools.partial(_k1_kernel, v, nchunk0, nj1),
        grid=(2, nj1),
        in_specs=[
            pl.BlockSpec((bk, dm), lambda r, j: (0, 0)),
            pl.BlockSpec((dm, td), lambda r, j: (0, 0)),
            pl.BlockSpec((1, td), lambda r, j: (0, 0)),
            pl.BlockSpec((_VB, td), lambda r, j: (r * nchunk0 + j, 0)),
        ],
        out_specs=[
            pl.BlockSpec((bk, _VB), lambda r, j: (0, r * nchunk0 + j)),
            pl.BlockSpec((1, bk, td), lambda r, j: (r, 0, 0)),
        ],
        out_shape=(
            jax.ShapeDtypeStruct((bk, v), jnp.float32),
            jax.ShapeDtypeStruct((2, bk, td), jnp.float32),
        ),
        scratch_shapes=[
            pltpu.VMEM((bk, td), jnp.bfloat16),
            pltpu.VMEM((bk, td), jnp.float32),
        ],
        compiler_params=pltpu.CompilerParams(
            dimension_semantics=("parallel", "arbitrary"),
            vmem_limit_bytes=64 * 1024 * 1024,
        ),
    )(kw2, W_proj, b2, W_emb)

    rs = min(64, bk // 2)
    n_strips = bk // rs
    prob, out = pl.pallas_call(
        functools.partial(_k2_kernel, v),
        grid=(2, n_strips // 2),
        in_specs=[
            pl.BlockSpec((rs, v),
                         lambda r, t: (r * (n_strips // 2) + t, 0)),
            pl.BlockSpec((2, rs, td),
                         lambda r, t: (0, r * (n_strips // 2) + t, 0)),
        ],
        out_specs=[
            pl.BlockSpec((rs, v),
                         lambda r, t: (r * (n_strips // 2) + t, 0)),
            pl.BlockSpec((rs, td),
                         lambda r, t: (r * (n_strips // 2) + t, 0)),
        ],
        out_shape=(
            jax.ShapeDtypeStruct((bk, v), jnp.float32),
            jax.ShapeDtypeStruct((bk, td), jnp.float32),
        ),
        compiler_params=pltpu.CompilerParams(
            dimension_semantics=("parallel", "arbitrary"),
            vmem_limit_bytes=64 * 1024 * 1024,
        ),
    )(cos, acc_part)

    return (out.reshape(bsz, kwn, td),
            prob.reshape(bsz, kwn, v),
            cos.reshape(bsz, kwn, v))
